# Initial kernel scaffold; baseline (speedup 1.0000x reference)
import jax
import jax.numpy as jnp
from jax.experimental import pallas as pl


def kernel(x1, edge_index1, batch1, x2, edge_index2, batch2, ccatp_feature, W1, b1, W2, b2, W3, b3, Wg1, bg1, Wg2, bg2, Wf1, bf1, bn1_g, bn1_b, Wf11, bf11, bn11_g, bn11_b, Wf2, bf2, bn2_g, bn2_b, Wo, bo):
    raise NotImplementedError("write your pallas kernel here")


# speedup vs baseline: 1.0303x; 1.0303x over previous
import functools

import jax
import jax.numpy as jnp
from jax.experimental import pallas as pl


def _rup(v, m):
    return (v + m - 1) // m * m


def _mm_kernel(x_ref, w_ref, b_ref, s_ref, t_ref, o_ref, *, relu):
    y = jnp.dot(x_ref[...], w_ref[...], preferred_element_type=jnp.float32)
    y = y + b_ref[...]
    if relu:
        y = jnp.maximum(y, 0.0)
    o_ref[...] = y * s_ref[...] + t_ref[...]


def _mm(x, w, b=None, relu=False, scale=None, shift=None):
    M, K = x.shape
    N = w.shape[1]
    bm = 512
    Kp = _rup(K, 128)
    Np = _rup(N, 128)
    bn = min(Np, 512)
    Mp = _rup(M, bm)

    xp = jnp.pad(x, ((0, Mp - M), (0, Kp - K)))
    wp = jnp.pad(w, ((0, Kp - K), (0, Np - N)))
    bv = jnp.zeros((N,), x.dtype) if b is None else b
    sv = jnp.ones((N,), x.dtype) if scale is None else scale
    tv = jnp.zeros((N,), x.dtype) if shift is None else shift
    bp = jnp.pad(bv, (0, Np - N)).reshape(1, Np)
    sp = jnp.pad(sv, (0, Np - N)).reshape(1, Np)
    tp = jnp.pad(tv, (0, Np - N)).reshape(1, Np)

    grid = (Mp // bm, Np // bn)
    out = pl.pallas_call(
        functools.partial(_mm_kernel, relu=relu),
        grid=grid,
        in_specs=[
            pl.BlockSpec((bm, Kp), lambda i, j: (i, 0)),
            pl.BlockSpec((Kp, bn), lambda i, j: (0, j)),
            pl.BlockSpec((1, bn), lambda i, j: (0, j)),
            pl.BlockSpec((1, bn), lambda i, j: (0, j)),
            pl.BlockSpec((1, bn), lambda i, j: (0, j)),
        ],
        out_specs=pl.BlockSpec((bm, bn), lambda i, j: (i, j)),
        out_shape=jax.ShapeDtypeStruct((Mp, Np), x.dtype),
    )(xp, wp, bp, sp, tp)
    return out[:M, :N]


def _gcn_conv(x, ei, W, b):
    n = x.shape[0]
    loop = jnp.arange(n, dtype=ei.dtype)
    src = jnp.concatenate([ei[0], loop])
    dst = jnp.concatenate([ei[1], loop])
    deg = jax.ops.segment_sum(jnp.ones_like(dst, dtype=x.dtype), dst, num_segments=n)
    dinv = jnp.where(deg > 0, deg ** -0.5, 0.0)
    norm = dinv[src] * dinv[dst]
    h = _mm(x, W)
    msg = h[src] * norm[:, None]
    return jax.ops.segment_sum(msg, dst, num_segments=n) + b


def _branch(x, ei, batch, W1, b1, W2, b2, W3, b3, Wg1, bg1, Wg2, bg2):
    B = 512
    h = jax.nn.relu(_gcn_conv(x, ei, W1, b1))
    h = jax.nn.relu(_gcn_conv(h, ei, W2, b2))
    h = jax.nn.relu(_gcn_conv(h, ei, W3, b3))
    g = jax.ops.segment_max(h, batch, num_segments=B)
    g = jnp.where(jnp.isfinite(g), g, 0.0)
    g = _mm(g, Wg1, b=bg1, relu=True)
    g = _mm(g, Wg2, b=bg2)
    return g


def kernel(x1, edge_index1, batch1, x2, edge_index2, batch2, ccatp_feature, W1, b1, W2, b2, W3, b3, Wg1, bg1, Wg2, bg2, Wf1, bf1, bn1_g, bn1_b, Wf11, bf11, bn11_g, bn11_b, Wf2, bf2, bn2_g, bn2_b, Wo, bo):
    h1 = _branch(x1, edge_index1, batch1, W1, b1, W2, b2, W3, b3, Wg1, bg1, Wg2, bg2)
    h2 = _branch(x2, edge_index2, batch2, W1, b1, W2, b2, W3, b3, Wg1, bg1, Wg2, bg2)
    xc = jnp.concatenate([h1, h2, ccatp_feature], axis=1)
    eps = 1e-5
    s = 1.0 / jnp.sqrt(1.0 + eps)
    xc = _mm(xc, Wf1, b=bf1, relu=True, scale=bn1_g * s, shift=bn1_b)
    xc = _mm(xc, Wf11, b=bf11, relu=True, scale=bn11_g * s, shift=bn11_b)
    xc = _mm(xc, Wf2, b=bf2, relu=True, scale=bn2_g * s, shift=bn2_b)
    return _mm(xc, Wo, b=bo)


# speedup vs baseline: 1.2095x; 1.1739x over previous
import functools

import jax
import jax.numpy as jnp
from jax.experimental import pallas as pl


def _rup(v, m):
    return (v + m - 1) // m * m


def _mm_kernel(x_ref, w_ref, b_ref, s_ref, t_ref, o_ref, *, relu):
    y = jnp.dot(x_ref[...], w_ref[...], preferred_element_type=jnp.float32)
    y = y + b_ref[...]
    if relu:
        y = jnp.maximum(y, 0.0)
    o_ref[...] = y * s_ref[...] + t_ref[...]


def _mm(x, w, b=None, relu=False, scale=None, shift=None):
    M, K = x.shape
    N = w.shape[1]
    bm = 512
    Kp = _rup(K, 128)
    Np = _rup(N, 128)
    bn = min(Np, 512)
    Mp = _rup(M, bm)

    xp = jnp.pad(x, ((0, Mp - M), (0, Kp - K)))
    wp = jnp.pad(w, ((0, Kp - K), (0, Np - N)))
    bv = jnp.zeros((N,), x.dtype) if b is None else b
    sv = jnp.ones((N,), x.dtype) if scale is None else scale
    tv = jnp.zeros((N,), x.dtype) if shift is None else shift
    bp = jnp.pad(bv, (0, Np - N)).reshape(1, Np)
    sp = jnp.pad(sv, (0, Np - N)).reshape(1, Np)
    tp = jnp.pad(tv, (0, Np - N)).reshape(1, Np)

    grid = (Mp // bm, Np // bn)
    out = pl.pallas_call(
        functools.partial(_mm_kernel, relu=relu),
        grid=grid,
        in_specs=[
            pl.BlockSpec((bm, Kp), lambda i, j: (i, 0)),
            pl.BlockSpec((Kp, bn), lambda i, j: (0, j)),
            pl.BlockSpec((1, bn), lambda i, j: (0, j)),
            pl.BlockSpec((1, bn), lambda i, j: (0, j)),
            pl.BlockSpec((1, bn), lambda i, j: (0, j)),
        ],
        out_specs=pl.BlockSpec((bm, bn), lambda i, j: (i, j)),
        out_shape=jax.ShapeDtypeStruct((Mp, Np), x.dtype),
    )(xp, wp, bp, sp, tp)
    return out[:M, :N]


def _gcn_conv(x, ei, W, b):
    n = x.shape[0]
    loop = jnp.arange(n, dtype=ei.dtype)
    src = jnp.concatenate([ei[0], loop])
    dst = jnp.concatenate([ei[1], loop])
    deg = jax.ops.segment_sum(jnp.ones_like(dst, dtype=x.dtype), dst, num_segments=n)
    dinv = jnp.where(deg > 0, deg ** -0.5, 0.0)
    norm = dinv[src] * dinv[dst]
    msg = x[src] * norm[:, None]
    agg = jax.ops.segment_sum(msg, dst, num_segments=n)
    return _mm(agg, W, b=b)


def _branch(x, ei, batch, W1, b1, W2, b2, W3, b3, Wg1, bg1, Wg2, bg2):
    B = 512
    h = jax.nn.relu(_gcn_conv(x, ei, W1, b1))
    h = jax.nn.relu(_gcn_conv(h, ei, W2, b2))
    h = jax.nn.relu(_gcn_conv(h, ei, W3, b3))
    g = jax.ops.segment_max(h, batch, num_segments=B)
    g = jnp.where(jnp.isfinite(g), g, 0.0)
    g = _mm(g, Wg1, b=bg1, relu=True)
    g = _mm(g, Wg2, b=bg2)
    return g


def kernel(x1, edge_index1, batch1, x2, edge_index2, batch2, ccatp_feature, W1, b1, W2, b2, W3, b3, Wg1, bg1, Wg2, bg2, Wf1, bf1, bn1_g, bn1_b, Wf11, bf11, bn11_g, bn11_b, Wf2, bf2, bn2_g, bn2_b, Wo, bo):
    h1 = _branch(x1, edge_index1, batch1, W1, b1, W2, b2, W3, b3, Wg1, bg1, Wg2, bg2)
    h2 = _branch(x2, edge_index2, batch2, W1, b1, W2, b2, W3, b3, Wg1, bg1, Wg2, bg2)
    xc = jnp.concatenate([h1, h2, ccatp_feature], axis=1)
    eps = 1e-5
    s = 1.0 / jnp.sqrt(1.0 + eps)
    xc = _mm(xc, Wf1, b=bf1, relu=True, scale=bn1_g * s, shift=bn1_b)
    xc = _mm(xc, Wf11, b=bf11, relu=True, scale=bn11_g * s, shift=bn11_b)
    xc = _mm(xc, Wf2, b=bf2, relu=True, scale=bn2_g * s, shift=bn2_b)
    return _mm(xc, Wo, b=bo)
